# baseline (speedup 1.0000x reference)
import functools

import jax
import jax.numpy as jnp
from jax import lax
from jax.experimental import pallas as pl
from jax.experimental.pallas import tpu as pltpu
from jax.experimental.pallas import tpu_sc as plsc

NTOK = 1000000
NHID = 64
NPAD = 128
BATCH = 4096
SEQ = 200
B = BATCH * SEQ

NUM_CORES = 2
NUM_SUBCORES = 16
NW = NUM_CORES * NUM_SUBCORES
PER_W = B // NW

_mesh = plsc.VectorSubcoreMesh(core_axis_name="c", subcore_axis_name="s")

FW = 256
NBLK_FULL = NTOK // FW
TAIL = NTOK - NBLK_FULL * FW
BLK_PER_W = 123


@functools.partial(
    pl.kernel,
    mesh=_mesh,
    out_type=jax.ShapeDtypeStruct((NTOK, NPAD), jnp.float32),
    scratch_types=[
        pltpu.VMEM((2, NHID, FW), jnp.float32),
        pltpu.VMEM((2, FW, NPAD), jnp.float32),
        pltpu.SemaphoreType.DMA,
        pltpu.SemaphoreType.DMA,
        pltpu.SemaphoreType.DMA,
        pltpu.SemaphoreType.DMA,
    ],
    compiler_params=pltpu.CompilerParams(
        use_tc_tiling_on_sc=True, needs_layout_passes=False
    ),
)
def _fmt(tbl_t_hbm, tail_hbm, out_hbm, vin, vout, i0, i1, o0, o1):
    isem = (i0, i1)
    osem = (o0, o1)
    wid = lax.axis_index("s") * NUM_CORES + lax.axis_index("c")
    c0 = wid * BLK_PER_W
    hi = jnp.minimum(c0 + BLK_PER_W, NBLK_FULL)
    row_ids = lax.iota(jnp.int32, 16)

    def fire_in(c, p):
        pltpu.async_copy(
            tbl_t_hbm.at[:, pl.ds(c * FW, FW)], vin.at[p], isem[p]
        )

    def wait_in(p):
        pltpu.make_async_copy(
            tbl_t_hbm.at[:, pl.ds(0, FW)], vin.at[p], isem[p]
        ).wait()

    def fire_out(c, p):
        pltpu.async_copy(
            vout.at[p], out_hbm.at[pl.ds(c * FW, FW)], osem[p]
        )

    def wait_out(p):
        pltpu.make_async_copy(
            vout.at[p], out_hbm.at[pl.ds(0, FW)], osem[p]
        ).wait()

    def transpose_block(p):
        @pl.loop(0, FW, unroll=16)
        def per_col(b):
            col_ids = jnp.full((16,), b, jnp.int32)
            for q in range(NHID // 16):
                v = plsc.load_gather(vin.at[p], [row_ids + q * 16, col_ids])
                vout[p, b, pl.ds(q * 16, 16)] = v

    @pl.when(c0 < hi)
    def _():
        fire_in(c0, 0)

    @pl.loop(0, (BLK_PER_W + 1) // 2)
    def outer(g):
        for par in range(2):
            c = c0 + g * 2 + par

            @pl.when(c + 1 < hi)
            def _():
                fire_in(c + 1, 1 - par)

            @pl.when(c < hi)
            def _():
                wait_in(par)

                @pl.when(c - c0 >= 2)
                def _():
                    wait_out(par)

                transpose_block(par)
                fire_out(c, par)

    cnt = hi - c0

    @pl.when(cnt >= 2)
    def _():
        wait_out(1)

    @pl.when(cnt >= 1)
    def _():
        wait_out(0)

    @pl.when(wid == NW - 1)
    def _():
        pltpu.sync_copy(tail_hbm, vout.at[0, pl.ds(0, TAIL)])
        pltpu.sync_copy(
            vout.at[0, pl.ds(0, TAIL)], out_hbm.at[pl.ds(NBLK_FULL * FW, TAIL)]
        )


GATHER = 128
N_G = 2
CHUNK = GATHER * N_G
N_CHUNKS = PER_W // CHUNK
N_IDX_ROWS = PER_W // GATHER
NBUF = 2


@functools.partial(
    pl.kernel,
    mesh=_mesh,
    out_type=jax.ShapeDtypeStruct((B, NPAD), jnp.float32),
    scratch_types=[
        pltpu.VMEM((N_IDX_ROWS, GATHER), jnp.int32),
        pltpu.VMEM((NBUF, CHUNK, NPAD), jnp.float32),
        pltpu.SemaphoreType.DMA,
        pltpu.SemaphoreType.DMA,
        pltpu.SemaphoreType.DMA,
        pltpu.SemaphoreType.DMA,
    ],
    compiler_params=pltpu.CompilerParams(use_tc_tiling_on_sc=False),
)
def _embed(seq_hbm, table_hbm, out_hbm, idx_v, rows_v, g0, g1, w0, w1):
    gsem = (g0, g1)
    wsem = (w0, w1)
    wid = lax.axis_index("s") * NUM_CORES + lax.axis_index("c")
    base = wid * PER_W
    row_base = pl.multiple_of(wid * N_IDX_ROWS, 8)

    pltpu.sync_copy(seq_hbm.at[pl.ds(row_base, N_IDX_ROWS)], idx_v)

    def fire(c, b):
        for g in range(N_G):
            pltpu.async_copy(
                table_hbm.at[idx_v.at[c * N_G + g]],
                rows_v.at[b, pl.ds(g * GATHER, GATHER)],
                gsem[b],
            )

    def drain_gather(b):
        pltpu.make_async_copy(
            out_hbm.at[pl.ds(0, CHUNK)], rows_v.at[b], gsem[b]
        ).wait()

    def start_write(c, b):
        pltpu.async_copy(
            rows_v.at[b], out_hbm.at[pl.ds(base + c * CHUNK, CHUNK)], wsem[b]
        )

    def drain_write(b):
        pltpu.make_async_copy(
            rows_v.at[b], out_hbm.at[pl.ds(0, CHUNK)], wsem[b]
        ).wait()

    fire(0, 0)

    @pl.loop(0, N_CHUNKS // NBUF)
    def outer(gidx):
        for b in range(NBUF):
            c = gidx * NBUF + b
            nb = (b + 1) % NBUF
            @pl.when(c + 1 < N_CHUNKS)
            def _():
                @pl.when(c + 1 >= NBUF)
                def _():
                    drain_write(nb)

                fire(c + 1, nb)

            drain_gather(b)
            start_write(c, b)

    drain_write((N_CHUNKS - 1) % NBUF)


def kernel(seq, table):
    tail128 = jnp.pad(table[NTOK - TAIL:], ((0, 0), (0, NPAD - NHID)))
    tblfmt = _fmt(table.T, tail128)
    seq2d = seq.reshape(B // GATHER, GATHER)
    out = _embed(seq2d, tblfmt)
    return out[:, :NHID].reshape(BATCH, SEQ, NHID)


# speedup vs baseline: 1.9601x; 1.9601x over previous
import functools

import jax
import jax.numpy as jnp
from jax import lax
from jax.experimental import pallas as pl
from jax.experimental.pallas import tpu as pltpu
from jax.experimental.pallas import tpu_sc as plsc

NTOK = 1000000
NHID = 64
NPAD = 128
BATCH = 4096
SEQ = 200
B = BATCH * SEQ

NUM_CORES = 2
NUM_SUBCORES = 16
NW = NUM_CORES * NUM_SUBCORES
PER_W = B // NW

GATHER = 128
N_G = 2
CHUNK = GATHER * N_G
N_CHUNKS = PER_W // CHUNK
N_IDX_ROWS = PER_W // GATHER
NBUF = 2

_mesh = plsc.VectorSubcoreMesh(core_axis_name="c", subcore_axis_name="s")


@functools.partial(
    pl.kernel,
    mesh=_mesh,
    out_type=jax.ShapeDtypeStruct((B, NPAD), jnp.float32),
    scratch_types=[
        pltpu.VMEM((N_IDX_ROWS, GATHER), jnp.int32),
        pltpu.VMEM((NBUF, CHUNK, NPAD), jnp.float32),
        pltpu.SemaphoreType.DMA,
        pltpu.SemaphoreType.DMA,
        pltpu.SemaphoreType.DMA,
        pltpu.SemaphoreType.DMA,
    ],
    compiler_params=pltpu.CompilerParams(use_tc_tiling_on_sc=False),
)
def _embed(seq_hbm, table_hbm, out_hbm, idx_v, rows_v, g0, g1, w0, w1):
    gsem = (g0, g1)
    wsem = (w0, w1)
    wid = lax.axis_index("s") * NUM_CORES + lax.axis_index("c")
    base = wid * PER_W
    row_base = pl.multiple_of(wid * N_IDX_ROWS, 8)

    pltpu.sync_copy(seq_hbm.at[pl.ds(row_base, N_IDX_ROWS)], idx_v)

    def fire(c, b):
        for g in range(N_G):
            pltpu.async_copy(
                table_hbm.at[idx_v.at[c * N_G + g]],
                rows_v.at[b, pl.ds(g * GATHER, GATHER)],
                gsem[b],
            )

    def drain_gather(b):
        pltpu.make_async_copy(
            out_hbm.at[pl.ds(0, CHUNK)], rows_v.at[b], gsem[b]
        ).wait()

    def start_write(c, b):
        pltpu.async_copy(
            rows_v.at[b], out_hbm.at[pl.ds(base + c * CHUNK, CHUNK)], wsem[b]
        )

    def drain_write(b):
        pltpu.make_async_copy(
            rows_v.at[b], out_hbm.at[pl.ds(0, CHUNK)], wsem[b]
        ).wait()

    fire(0, 0)

    @pl.loop(0, N_CHUNKS // NBUF)
    def outer(gidx):
        for b in range(NBUF):
            c = gidx * NBUF + b
            nb = (b + 1) % NBUF
            @pl.when(c + 1 < N_CHUNKS)
            def _():
                @pl.when(c + 1 >= NBUF)
                def _():
                    drain_write(nb)

                fire(c + 1, nb)

            drain_gather(b)
            start_write(c, b)

    drain_write((N_CHUNKS - 1) % NBUF)


def kernel(seq, table):
    seq2d = seq.reshape(B // GATHER, GATHER)
    tbl128 = jnp.pad(table, ((0, 0), (0, NPAD - NHID)))
    out = _embed(seq2d, tbl128)
    return out[:, :NHID].reshape(BATCH, SEQ, NHID)


# speedup vs baseline: 2.1101x; 1.0765x over previous
import functools

import jax
import jax.numpy as jnp
from jax import lax
from jax.experimental import pallas as pl
from jax.experimental.pallas import tpu as pltpu
from jax.experimental.pallas import tpu_sc as plsc

NTOK = 1000000
NHID = 64
NPAD = 128
BATCH = 4096
SEQ = 200
B = BATCH * SEQ

NUM_CORES = 2
NUM_SUBCORES = 16
NW = NUM_CORES * NUM_SUBCORES
PER_W = B // NW

GATHER = 128
N_G = 2
CHUNK = GATHER * N_G
N_CHUNKS = PER_W // CHUNK
N_IDX_ROWS = PER_W // GATHER
NBUF = 2

_mesh = plsc.VectorSubcoreMesh(core_axis_name="c", subcore_axis_name="s")


@functools.partial(
    pl.kernel,
    mesh=_mesh,
    out_type=jax.ShapeDtypeStruct((B, NPAD), jnp.float32),
    scratch_types=[
        pltpu.VMEM((N_IDX_ROWS, GATHER), jnp.int32),
        pltpu.VMEM((NBUF, CHUNK, NPAD), jnp.float32),
        pltpu.SemaphoreType.DMA,
        pltpu.SemaphoreType.DMA,
        pltpu.SemaphoreType.DMA,
        pltpu.SemaphoreType.DMA,
    ],
    compiler_params=pltpu.CompilerParams(use_tc_tiling_on_sc=False),
)
def _embed(seq_hbm, table_hbm, out_hbm, idx_v, rows_v, g0, g1, w0, w1):
    gsem = (g0, g1)
    wsem = (w0, w1)
    wid = lax.axis_index("s") * NUM_CORES + lax.axis_index("c")
    base = wid * PER_W
    row_base = pl.multiple_of(wid * N_IDX_ROWS, 8)

    pltpu.sync_copy(seq_hbm.at[pl.ds(row_base, N_IDX_ROWS)], idx_v)

    def fire(c, b):
        for g in range(N_G):
            pltpu.async_copy(
                table_hbm.at[idx_v.at[c * N_G + g]],
                rows_v.at[b, pl.ds(g * GATHER, GATHER)],
                gsem[b],
            )

    def drain_gather(b):
        pltpu.make_async_copy(
            out_hbm.at[pl.ds(0, CHUNK)], rows_v.at[b], gsem[b]
        ).wait()

    def start_write(c, b):
        pltpu.async_copy(
            rows_v.at[b, pl.ds(0, CHUNK), pl.ds(0, NHID)],
            out_hbm.at[pl.ds(base + c * CHUNK, CHUNK), pl.ds(0, NHID)],
            wsem[b],
        )

    def drain_write(b):
        pltpu.make_async_copy(
            rows_v.at[b, pl.ds(0, CHUNK), pl.ds(0, NHID)],
            out_hbm.at[pl.ds(0, CHUNK), pl.ds(0, NHID)],
            wsem[b],
        ).wait()

    fire(0, 0)

    @pl.loop(0, N_CHUNKS // NBUF)
    def outer(gidx):
        for b in range(NBUF):
            c = gidx * NBUF + b
            nb = (b + 1) % NBUF
            @pl.when(c + 1 < N_CHUNKS)
            def _():
                @pl.when(c + 1 >= NBUF)
                def _():
                    drain_write(nb)

                fire(c + 1, nb)

            drain_gather(b)
            start_write(c, b)

    drain_write((N_CHUNKS - 1) % NBUF)


def kernel(seq, table):
    seq2d = seq.reshape(B // GATHER, GATHER)
    tbl128 = jnp.pad(table, ((0, 0), (0, NPAD - NHID)))
    out = _embed(seq2d, tbl128)
    return out[:, :NHID].reshape(BATCH, SEQ, NHID)
